# baseline (speedup 1.0000x reference)
import functools

import jax
import jax.numpy as jnp
from jax import lax
from jax.experimental import pallas as pl
from jax.experimental.pallas import tpu as pltpu
from jax.experimental.pallas import tpu_sc as plsc

_NUM_SCRATCHES = 20
_MAX_LENGTH = 50
_COLOR = 1.0
_NC, _NS = 2, 16
_NW = _NC * _NS
_LANES = 128


def _scratch_points(N, H, W):
    key = jax.random.key(42)
    k1, k2, k3, k4 = jax.random.split(key, 4)
    x_start = jax.random.randint(k1, (N, _NUM_SCRATCHES), 0, W)
    y_start = jax.random.randint(k2, (N, _NUM_SCRATCHES), 0, H)
    lengths = jax.random.randint(k3, (N, _NUM_SCRATCHES), 1, _MAX_LENGTH + 1)
    lengths = lengths.astype(jnp.float32)
    angles = jax.random.uniform(k4, (N, _NUM_SCRATCHES)) * 2 * 3.14159
    x_end = x_start.astype(jnp.float32) + lengths * jnp.cos(angles)
    y_end = y_start.astype(jnp.float32) + lengths * jnp.sin(angles)
    steps = int(_MAX_LENGTH * 1.5)
    t = jnp.linspace(0.0, 1.0, steps).reshape(1, 1, steps)
    xs = x_start.astype(jnp.float32)[..., None]
    ys = y_start.astype(jnp.float32)[..., None]
    xe = x_end[..., None]
    ye = y_end[..., None]
    x_points = (xs * (1 - t) + xe * t).astype(jnp.int32)
    y_points = (ys * (1 - t) + ye * t).astype(jnp.int32)
    x_points = jnp.clip(x_points, 0, W - 1).reshape(N, -1)
    y_points = jnp.clip(y_points, 0, H - 1).reshape(N, -1)
    return x_points, y_points


def _flat_indices(N, C, H, W):
    xp, yp = _scratch_points(N, H, W)
    P = xp.shape[1]
    x1 = jnp.clip(xp + 1, 0, W - 1)
    y1 = jnp.clip(yp + 1, 0, H - 1)
    pix = jnp.stack([yp * W + xp, y1 * W + xp, yp * W + x1], 1)
    pix = pix.reshape(N, 3 * P)
    base = (jnp.arange(N) * C)[:, None, None] * (H * W)
    chan = (jnp.arange(C) * (H * W))[None, :, None]
    idx = (base + chan + pix[:, None, :]).reshape(N, C * 3 * P)
    K = idx.shape[1]
    chunks = -(-K // _LANES)
    pad = chunks * _LANES - K
    if pad:
        idx = jnp.concatenate([idx, jnp.broadcast_to(idx[:, :1], (N, pad))], 1)
    return idx.astype(jnp.int32)


def kernel(img):
    N, C, H, W = img.shape
    size = N * C * H * W
    assert N == _NW, "one worker per image"
    per_w = size // _NW

    idx = _flat_indices(N, C, H, W)
    nidx = idx.shape[1]
    color = jnp.full((nidx,), _COLOR, jnp.float32)
    flat = img.reshape(size)

    NBUF = 4
    PRIME = 2
    CHUNK = 24576
    nchunks = per_w // CHUNK
    assert nchunks * CHUNK == per_w

    mesh = plsc.VectorSubcoreMesh(core_axis_name="c", subcore_axis_name="s")

    @functools.partial(
        pl.kernel,
        out_type=jax.ShapeDtypeStruct((size,), jnp.float32),
        mesh=mesh,
        scratch_types=[
            pltpu.VMEM((nidx,), jnp.int32),
            pltpu.VMEM((nidx,), jnp.float32),
            [pltpu.VMEM((CHUNK,), jnp.float32)] * NBUF,
            [pltpu.SemaphoreType.DMA] * NBUF,
            [pltpu.SemaphoreType.DMA] * NBUF,
            pltpu.SemaphoreType.DMA,
            pltpu.SemaphoreType.DMA,
        ],
    )
    def scratches_sc(img_hbm, idx_hbm, color_hbm, out_hbm,
                     idx_v, color_v, bufs, in_sems, out_sems, stage_sem,
                     scat_sem):
        wid = lax.axis_index("s") * _NC + lax.axis_index("c")
        base = wid * per_w
        idx_cp = pltpu.async_copy(idx_hbm.at[wid], idx_v, stage_sem)
        col_cp = pltpu.async_copy(color_hbm, color_v, stage_sem)

        def chunk(g):
            return pl.ds(base + g * CHUNK, CHUNK)

        for b in range(min(PRIME, nchunks)):
            pltpu.async_copy(img_hbm.at[chunk(b)], bufs[b], in_sems[b])
        for g in range(nchunks):
            b = g % NBUF
            pltpu.make_async_copy(img_hbm.at[chunk(g)], bufs[b],
                                  in_sems[b]).wait()
            pltpu.async_copy(bufs[b], out_hbm.at[chunk(g)], out_sems[b])
            p = g + PRIME
            if p < nchunks:
                pb = p % NBUF
                if p >= NBUF:
                    pltpu.make_async_copy(bufs[pb],
                                          out_hbm.at[chunk(p - NBUF)],
                                          out_sems[pb]).wait()
                pltpu.async_copy(img_hbm.at[chunk(p)], bufs[pb], in_sems[pb])
        for g in range(max(0, nchunks - NBUF), nchunks):
            b = g % NBUF
            pltpu.make_async_copy(bufs[b], out_hbm.at[chunk(g)],
                                  out_sems[b]).wait()

        idx_cp.wait()
        col_cp.wait()
        pltpu.async_copy(color_v, out_hbm.at[idx_v], scat_sem).wait()

    return scratches_sc(flat, idx, color).reshape(N, C, H, W)


# speedup vs baseline: 1.0047x; 1.0047x over previous
import functools

import jax
import jax.numpy as jnp
from jax import lax
from jax.experimental import pallas as pl
from jax.experimental.pallas import tpu as pltpu
from jax.experimental.pallas import tpu_sc as plsc

_NUM_SCRATCHES = 20
_MAX_LENGTH = 50
_COLOR = 1.0
_NC, _NS = 2, 16
_NW = _NC * _NS
_LANES = 128


def _scratch_points(N, H, W):
    key = jax.random.key(42)
    k1, k2, k3, k4 = jax.random.split(key, 4)
    x_start = jax.random.randint(k1, (N, _NUM_SCRATCHES), 0, W)
    y_start = jax.random.randint(k2, (N, _NUM_SCRATCHES), 0, H)
    lengths = jax.random.randint(k3, (N, _NUM_SCRATCHES), 1, _MAX_LENGTH + 1)
    lengths = lengths.astype(jnp.float32)
    angles = jax.random.uniform(k4, (N, _NUM_SCRATCHES)) * 2 * 3.14159
    x_end = x_start.astype(jnp.float32) + lengths * jnp.cos(angles)
    y_end = y_start.astype(jnp.float32) + lengths * jnp.sin(angles)
    steps = int(_MAX_LENGTH * 1.5)
    t = jnp.linspace(0.0, 1.0, steps).reshape(1, 1, steps)
    xs = x_start.astype(jnp.float32)[..., None]
    ys = y_start.astype(jnp.float32)[..., None]
    xe = x_end[..., None]
    ye = y_end[..., None]
    x_points = (xs * (1 - t) + xe * t).astype(jnp.int32)
    y_points = (ys * (1 - t) + ye * t).astype(jnp.int32)
    x_points = jnp.clip(x_points, 0, W - 1).reshape(N, -1)
    y_points = jnp.clip(y_points, 0, H - 1).reshape(N, -1)
    return x_points, y_points


def _flat_indices(N, C, H, W):
    xp, yp = _scratch_points(N, H, W)
    P = xp.shape[1]
    x1 = jnp.clip(xp + 1, 0, W - 1)
    y1 = jnp.clip(yp + 1, 0, H - 1)
    pix = jnp.stack([yp * W + xp, y1 * W + xp, yp * W + x1], 1)
    pix = pix.reshape(N, 3 * P)
    base = (jnp.arange(N) * C)[:, None, None] * (H * W)
    chan = (jnp.arange(C) * (H * W))[None, :, None]
    idx = (base + chan + pix[:, None, :]).reshape(N, C * 3 * P)
    K = idx.shape[1]
    chunks = -(-K // _LANES)
    pad = chunks * _LANES - K
    if pad:
        idx = jnp.concatenate([idx, jnp.broadcast_to(idx[:, :1], (N, pad))], 1)
    return idx.astype(jnp.int32)


def kernel(img):
    N, C, H, W = img.shape
    size = N * C * H * W
    assert N == _NW, "one worker per image"
    per_w = size // _NW

    idx = _flat_indices(N, C, H, W)
    nidx = idx.shape[1]
    color = jnp.full((nidx,), _COLOR, jnp.float32)
    flat = img.reshape(size)

    NBUF = 2
    PRIME = 1
    CHUNK = 49152
    nchunks = per_w // CHUNK
    assert nchunks * CHUNK == per_w

    mesh = plsc.VectorSubcoreMesh(core_axis_name="c", subcore_axis_name="s")

    @functools.partial(
        pl.kernel,
        out_type=jax.ShapeDtypeStruct((size,), jnp.float32),
        mesh=mesh,
        scratch_types=[
            pltpu.VMEM((nidx,), jnp.int32),
            pltpu.VMEM((nidx,), jnp.float32),
            pltpu.VMEM_SHARED((NBUF, _NS, CHUNK), jnp.float32),
            [pltpu.SemaphoreType.DMA] * NBUF,
            [pltpu.SemaphoreType.DMA] * NBUF,
            pltpu.SemaphoreType.DMA,
            pltpu.SemaphoreType.DMA,
        ],
    )
    def scratches_sc(img_hbm, idx_hbm, color_hbm, out_hbm,
                     idx_v, color_v, spbuf, in_sems, out_sems, stage_sem,
                     scat_sem):
        sid = lax.axis_index("s")
        wid = sid * _NC + lax.axis_index("c")
        base = wid * per_w
        bufs = [spbuf.at[b, sid] for b in range(NBUF)]
        idx_cp = pltpu.async_copy(idx_hbm.at[wid], idx_v, stage_sem)
        col_cp = pltpu.async_copy(color_hbm, color_v, stage_sem)

        def chunk(g):
            return pl.ds(base + g * CHUNK, CHUNK)

        for b in range(min(PRIME, nchunks)):
            pltpu.async_copy(img_hbm.at[chunk(b)], bufs[b], in_sems[b])
        for g in range(nchunks):
            b = g % NBUF
            pltpu.make_async_copy(img_hbm.at[chunk(g)], bufs[b],
                                  in_sems[b]).wait()
            pltpu.async_copy(bufs[b], out_hbm.at[chunk(g)], out_sems[b])
            p = g + PRIME
            if p < nchunks:
                pb = p % NBUF
                if p >= NBUF:
                    pltpu.make_async_copy(bufs[pb],
                                          out_hbm.at[chunk(p - NBUF)],
                                          out_sems[pb]).wait()
                pltpu.async_copy(img_hbm.at[chunk(p)], bufs[pb], in_sems[pb])
        for g in range(max(0, nchunks - NBUF), nchunks):
            b = g % NBUF
            pltpu.make_async_copy(bufs[b], out_hbm.at[chunk(g)],
                                  out_sems[b]).wait()

        idx_cp.wait()
        col_cp.wait()
        pltpu.async_copy(color_v, out_hbm.at[idx_v], scat_sem).wait()

    return scratches_sc(flat, idx, color).reshape(N, C, H, W)


# speedup vs baseline: 9.3048x; 9.2615x over previous
import functools

import jax
import jax.numpy as jnp
from jax import lax
from jax.experimental import pallas as pl
from jax.experimental.pallas import tpu as pltpu

_NUM_SCRATCHES = 20
_MAX_LENGTH = 50
_COLOR = 1.0


def _scratch_points(N, H, W):
    key = jax.random.key(42)
    k1, k2, k3, k4 = jax.random.split(key, 4)
    x_start = jax.random.randint(k1, (N, _NUM_SCRATCHES), 0, W)
    y_start = jax.random.randint(k2, (N, _NUM_SCRATCHES), 0, H)
    lengths = jax.random.randint(k3, (N, _NUM_SCRATCHES), 1, _MAX_LENGTH + 1)
    lengths = lengths.astype(jnp.float32)
    angles = jax.random.uniform(k4, (N, _NUM_SCRATCHES)) * 2 * 3.14159
    x_end = x_start.astype(jnp.float32) + lengths * jnp.cos(angles)
    y_end = y_start.astype(jnp.float32) + lengths * jnp.sin(angles)
    steps = int(_MAX_LENGTH * 1.5)
    t = jnp.linspace(0.0, 1.0, steps).reshape(1, 1, steps)
    xs = x_start.astype(jnp.float32)[..., None]
    ys = y_start.astype(jnp.float32)[..., None]
    xe = x_end[..., None]
    ye = y_end[..., None]
    x_points = (xs * (1 - t) + xe * t).astype(jnp.int32)
    y_points = (ys * (1 - t) + ye * t).astype(jnp.int32)
    x_points = jnp.clip(x_points, 0, W - 1).reshape(N, -1)
    y_points = jnp.clip(y_points, 0, H - 1).reshape(N, -1)
    return x_points, y_points


@functools.cache
def _mask_const(N, H, W):
    with jax.ensure_compile_time_eval():
        xp, yp = _scratch_points(N, H, W)
        x1 = jnp.clip(xp + 1, 0, W - 1)
        y1 = jnp.clip(yp + 1, 0, H - 1)
        n = jnp.broadcast_to(jnp.arange(N)[:, None], xp.shape)
        flat = jnp.concatenate([
            (n * H + yp) * W + xp,
            (n * H + y1) * W + xp,
            (n * H + yp) * W + x1,
        ], axis=1).reshape(-1)
        mask = jnp.zeros((N * H * W,), jnp.uint8).at[flat].set(1)
        return mask.reshape(N, H, W)


def kernel(img):
    N, C, H, W = img.shape
    mask = _mask_const(N, H, W)

    def body(img_ref, mask_ref, out_ref):
        m = mask_ref[0] != 0
        out_ref[0, 0] = jnp.where(m, jnp.float32(_COLOR), img_ref[0, 0])

    return pl.pallas_call(
        body,
        grid=(N, C),
        in_specs=[
            pl.BlockSpec((1, 1, H, W), lambda n, c: (n, c, 0, 0)),
            pl.BlockSpec((1, H, W), lambda n, c: (n, 0, 0)),
        ],
        out_specs=pl.BlockSpec((1, 1, H, W), lambda n, c: (n, c, 0, 0)),
        out_shape=jax.ShapeDtypeStruct((N, C, H, W), jnp.float32),
    )(img, mask)
